# Initial kernel scaffold; baseline (speedup 1.0000x reference)
import jax
import jax.numpy as jnp
from jax.experimental import pallas as pl


def kernel(x, edge_index, edge_attr, batch, emb, W1, b1, W2, b2, p1, p2, gW, gb, pW1, pb1, pW2, pb2):
    raise NotImplementedError("write your pallas kernel here")


# speedup vs baseline: 10.9795x; 10.9795x over previous
import functools
import numpy as np
import jax
import jax.numpy as jnp
from jax import lax
from jax.experimental import pallas as pl
from jax.experimental.pallas import tpu as pltpu
from jax.experimental.pallas import tpu_sc as plsc

N = 10000
E = 320000
H = 128
K1 = 5000
K2 = 2500
NPAD = 10240
ROWS = NPAD // 128
NEG_INF = np.float32(-np.inf)
I32_MIN = np.int32(-(2 ** 31))

NC, NS, LANES = 2, 16, 16
NW = NC * NS
EPW = E // NW
ECH = 128
NFULL = EPW // ECH
ETAIL = EPW - NFULL * ECH
NODES_PW = NPAD // NW
NCHUNK = 8
IDXC = NCHUNK * 16
L_TOK = 16


def _mesh():
    return plsc.VectorSubcoreMesh(core_axis_name="c", subcore_axis_name="s",
                                  num_cores=NC, num_subcores=NS)


def _sc_embed_body(emb_hbm, xflat_hbm, h0_hbm, idx_v, rows_v, hbuf_v, sem):
    wid = lax.axis_index("s") * NC + lax.axis_index("c")
    nbase = wid * NODES_PW

    def chunk(ci, carry):
        node0 = nbase + ci * NCHUNK
        pltpu.sync_copy(xflat_hbm.at[pl.ds(node0 * L_TOK, IDXC)], idx_v)
        pltpu.async_copy(emb_hbm.at[idx_v], rows_v, sem).wait()

        def node(n, carry2):
            for j in range(H // 16):
                acc = rows_v[n * L_TOK, pl.ds(j * 16, 16)]
                for l in range(1, L_TOK):
                    acc = acc + rows_v[n * L_TOK + l, pl.ds(j * 16, 16)]
                hbuf_v[n, pl.ds(j * 16, 16)] = acc
            return carry2

        lax.fori_loop(0, NCHUNK, node, 0)
        pltpu.sync_copy(hbuf_v, h0_hbm.at[pl.ds(node0, NCHUNK)])
        return carry

    lax.fori_loop(0, NPAD // NW // NCHUNK, chunk, 0)


def _sc_embed(embz, xflat):
    return pl.kernel(
        _sc_embed_body,
        out_type=jax.ShapeDtypeStruct((NPAD, H), jnp.float32),
        mesh=_mesh(),
        scratch_types=[
            pltpu.VMEM((IDXC,), jnp.int32),
            pltpu.VMEM((IDXC, H), jnp.float32),
            pltpu.VMEM((NCHUNK, H), jnp.float32),
            pltpu.SemaphoreType.DMA,
        ],
    )(embz, xflat)


def _lane_splat(v16, lane):
    dnums = lax.GatherDimensionNumbers(offset_dims=(),
                                       collapsed_slice_dims=(0,),
                                       start_index_map=(0,))
    idx = jnp.full((16, 1), lane, jnp.int32)
    return lax.gather(v16, idx, dnums, slice_sizes=(1,),
                      mode=lax.GatherScatterMode.PROMISE_IN_BOUNDS)


def _sc_scatter_body(xwd_hbm, row_hbm, col_hbm, ww_hbm, zeros_hbm, out_hbm,
                     ridx_v, cidx_v, ww_v, rows_v,
                     ridx_t, cidx_t, ww_t, rows_t,
                     acc_sh, sem):
    cid = lax.axis_index("c")
    sid = lax.axis_index("s")
    wid = sid * NC + cid
    ebase = wid * EPW
    nps = NPAD // NS
    pltpu.sync_copy(zeros_hbm.at[pl.ds(sid * nps, nps)],
                    acc_sh.at[pl.ds(sid * nps, nps)])
    plsc.subcore_barrier()

    def process(size, ridx, cidx, wwv, rows, e0):
        pltpu.sync_copy(row_hbm.at[pl.ds(e0, size)], ridx)
        pltpu.sync_copy(col_hbm.at[pl.ds(e0, size)], cidx)
        pltpu.sync_copy(ww_hbm.at[pl.ds(e0, size)], wwv)
        pltpu.async_copy(xwd_hbm.at[ridx], rows, sem).wait()

        def scale(g, c):
            e16 = wwv[pl.ds(g * 16, 16)]
            for l in range(16):
                w16 = _lane_splat(e16, l)
                e = g * 16 + l
                for j in range(H // 16):
                    rows[e, pl.ds(j * 16, 16)] = (
                        rows[e, pl.ds(j * 16, 16)] * w16)
            return c

        lax.fori_loop(0, size // 16, scale, 0)
        pltpu.sync_copy(rows, acc_sh.at[cidx], add=True)

    def chunk(i, c):
        process(ECH, ridx_v, cidx_v, ww_v, rows_v, ebase + i * ECH)
        return c

    lax.fori_loop(0, NFULL, chunk, 0)
    process(ETAIL, ridx_t, cidx_t, ww_t, rows_t, ebase + NFULL * ECH)
    plsc.subcore_barrier()
    pltpu.sync_copy(acc_sh.at[pl.ds(sid * nps, nps)],
                    out_hbm.at[cid, pl.ds(sid * nps, nps)])


def _sc_scatter(xwd, row, col, ww, zeros_nh):
    return pl.kernel(
        _sc_scatter_body,
        out_type=jax.ShapeDtypeStruct((NC, NPAD, H), jnp.float32),
        mesh=_mesh(),
        scratch_types=[
            pltpu.VMEM((ECH,), jnp.int32),
            pltpu.VMEM((ECH,), jnp.int32),
            pltpu.VMEM((ECH,), jnp.float32),
            pltpu.VMEM((ECH, H), jnp.float32),
            pltpu.VMEM((ETAIL,), jnp.int32),
            pltpu.VMEM((ETAIL,), jnp.int32),
            pltpu.VMEM((ETAIL,), jnp.float32),
            pltpu.VMEM((ETAIL, H), jnp.float32),
            pltpu.VMEM_SHARED((NPAD, H), jnp.float32),
            pltpu.SemaphoreType.DMA,
        ],
    )(xwd, row, col, ww, zeros_nh)


def _tokey(f):
    b = jax.lax.bitcast_convert_type(f, jnp.int32)
    return jnp.where(b >= 0, b, (~b) ^ I32_MIN)


def _fromkey(k):
    b = jnp.where(k >= 0, k, (~k) ^ I32_MIN)
    return jax.lax.bitcast_convert_type(b, jnp.float32)


def _threshold_mask(sc, valid, k):
    scp = jnp.concatenate([jnp.where(valid, sc, NEG_INF),
                           jnp.full((NPAD - N,), NEG_INF, jnp.float32)])
    vp = jnp.concatenate([valid.astype(jnp.float32),
                          jnp.zeros((NPAD - N,), jnp.float32)]) > 0
    keys = _tokey(scp)
    kmin = jnp.min(jnp.where(vp, keys, jnp.int32(2 ** 31 - 1)))
    kmax = jnp.max(jnp.where(vp, keys, I32_MIN))

    def body(_, lohi):
        lo, hi = lohi
        mid = lo + (hi - lo + 1) // 2
        t = _fromkey(mid)
        cnt = jnp.sum(jnp.where(vp & (scp >= t), 1, 0))
        pred = cnt >= k
        return (jnp.where(pred, mid, lo), jnp.where(pred, hi, mid - 1))

    lo, _ = jax.lax.fori_loop(0, 32, body, (kmin, kmax))
    t = _fromkey(lo)
    gt = vp & (scp > t)
    c = jnp.sum(jnp.where(gt, 1, 0))
    eq = vp & (scp == t)
    eqf = eq.astype(jnp.float32).reshape(ROWS, 128)
    ii = jax.lax.broadcasted_iota(jnp.int32, (128, 128), 0)
    jj = jax.lax.broadcasted_iota(jnp.int32, (128, 128), 1)
    U = (ii <= jj).astype(jnp.float32)
    bi = jax.lax.broadcasted_iota(jnp.int32, (ROWS, ROWS), 0)
    bj = jax.lax.broadcasted_iota(jnp.int32, (ROWS, ROWS), 1)
    Ls = (bj < bi).astype(jnp.float32)
    within = jnp.dot(eqf, U, preferred_element_type=jnp.float32)
    rowsum = jnp.sum(eqf, axis=1, keepdims=True)
    prefix = jnp.dot(Ls, rowsum, preferred_element_type=jnp.float32)
    rank = (within + prefix).reshape(NPAD)
    need = (k - c).astype(jnp.float32)
    mask = gt | (eq & (rank <= need))
    return mask[:N]


def _attpool(xsel, mask, gW, gb):
    gate = jnp.dot(xsel, gW, preferred_element_type=jnp.float32)[:, 0] + gb[0]
    gate = jnp.where(mask, gate, NEG_INF)
    mx = jnp.max(gate)
    a = jnp.where(mask, jnp.exp(gate - mx), 0.0)
    a = a / jnp.sum(a)
    return jnp.dot(a[None, :], xsel, preferred_element_type=jnp.float32)


def _ew_body(attr_ref, x_ref, out_ref, inv_ref):
    a = attr_ref[...]
    ii = jax.lax.broadcasted_iota(jnp.int32, (128, 32), 0)
    jj = jax.lax.broadcasted_iota(jnp.int32, (128, 32), 1)
    Q = jnp.where((ii // 4) == jj, jnp.float32(0.25), jnp.float32(0.0))
    out_ref[...] = jnp.dot(a, Q, preferred_element_type=jnp.float32)
    cnt = jnp.sum((x_ref[...] != 0).astype(jnp.float32), axis=1,
                  keepdims=True)
    inv_ref[...] = 1.0 / jnp.maximum(cnt, 1.0)


def _gcnprep_body(h_ref, inv_ref, W_ref, degp_ref, wf_ref, xwd_ref, dis_ref):
    d3 = degp_ref[...]
    wf = wf_ref[...]
    d = wf * (d3[0, :N, 0:1] + d3[1, :N, 0:1]) + wf
    dis = jnp.where(d > 0, jax.lax.rsqrt(jnp.where(d > 0, d, 1.0)), 0.0)
    xw = jnp.dot(h_ref[...] * inv_ref[...], W_ref[...],
                 preferred_element_type=jnp.float32)
    xwd_ref[...] = xw * dis * wf
    dis_ref[...] = dis


def _hsc_body(s_ref, xwd_ref, dis_ref, wf_ref, b_ref, p_ref, h_ref, sc_ref):
    s = s_ref[0, :N, :] + s_ref[1, :N, :]
    h = jax.nn.relu(dis_ref[...] * (wf_ref[...] * s + xwd_ref[...])
                    + b_ref[...])
    p = p_ref[...]
    pn = p * jax.lax.rsqrt(jnp.sum(p * p))
    h_ref[...] = h
    sc_ref[...] = jnp.dot(h, pn, preferred_element_type=jnp.float32)


def _pool1_body(h_ref, sc_ref, gW_ref, gb_ref, W2_ref,
                xw2_ref, wfac_ref, out1_ref):
    h = h_ref[...]
    sc = sc_ref[...][:, 0]
    mask = _threshold_mask(sc, jnp.ones((N,), jnp.bool_), K1)
    wf = mask.astype(jnp.float32)[:, None]
    xsel = h * jnp.tanh(sc)[:, None] * wf
    out1_ref[...] = _attpool(xsel, mask, gW_ref[...], gb_ref[...])
    xw2_ref[...] = jnp.dot(xsel, W2_ref[...], preferred_element_type=jnp.float32)
    wfac_ref[...] = wf


def _pool2_body(h_ref, sc_ref, wfac_ref, gW_ref, gb_ref, out1_ref,
                pW1_ref, pb1_ref, pW2_ref, pb2_ref,
                logits_ref, act_ref):
    h2 = h_ref[...]
    sc2 = sc_ref[...][:, 0]
    valid = wfac_ref[...][:, 0] > 0
    mask2 = _threshold_mask(sc2, valid, K2)
    xsel2 = h2 * jnp.tanh(sc2)[:, None] * mask2.astype(jnp.float32)[:, None]
    out2 = _attpool(xsel2, mask2, gW_ref[...], gb_ref[...])
    act = out1_ref[...] + out2
    z1 = jax.nn.relu(jnp.dot(act, pW1_ref[...],
                             preferred_element_type=jnp.float32) + pb1_ref[...])
    z = jnp.dot(z1, pW2_ref[...], preferred_element_type=jnp.float32) + pb2_ref[...]
    logits_ref[...] = z * jax.lax.rsqrt(jnp.sum(z * z, axis=-1, keepdims=True))
    act_ref[...] = act


def _tc(body, out_shapes):
    return pl.pallas_call(body, out_shape=out_shapes)


def kernel(x, edge_index, edge_attr, batch, emb, W1, b1, W2, b2, p1, p2,
           gW, gb, pW1, pb1, pW2, pb2):
    row = edge_index[0]
    col = edge_index[1]

    attr2d = edge_attr.reshape(E // 32, 128)
    xpad = jnp.pad(x, ((0, NPAD - N), (0, 0)))
    ew2d, inv = _tc(_ew_body,
                    (jax.ShapeDtypeStruct((E // 32, 32), jnp.float32),
                     jax.ShapeDtypeStruct((NPAD, 1), jnp.float32)))(
        attr2d, xpad)
    ew = ew2d.reshape(E)

    embz = emb.at[0].set(0.0)
    xflat = xpad.reshape(NPAD * L_TOK)
    h0 = _sc_embed(embz, xflat)[:N]

    zeros_nh = jnp.zeros((NPAD, H), jnp.float32)
    ones_nh = jnp.ones((N, H), jnp.float32)
    degp1 = _sc_scatter(ones_nh, row, col, ew, zeros_nh)
    ones_col = jnp.ones((N, 1), jnp.float32)
    xwd1, dis1 = _tc(_gcnprep_body,
                     (jax.ShapeDtypeStruct((N, H), jnp.float32),
                      jax.ShapeDtypeStruct((N, 1), jnp.float32)))(
        h0, inv[:N], W1, degp1, ones_col)
    s1 = _sc_scatter(xwd1, row, col, ew, zeros_nh)

    h1, sc1 = _tc(_hsc_body,
                  (jax.ShapeDtypeStruct((N, H), jnp.float32),
                   jax.ShapeDtypeStruct((N, 1), jnp.float32)))(
        s1, xwd1, dis1, ones_col, b1[None, :], p1[:, None])
    xw2, wfac, out1 = _tc(_pool1_body,
                          (jax.ShapeDtypeStruct((N, H), jnp.float32),
                           jax.ShapeDtypeStruct((N, 1), jnp.float32),
                           jax.ShapeDtypeStruct((1, H), jnp.float32)))(
        h1, sc1, gW, gb, W2)

    wf_tab = jnp.tile(wfac, (1, H))
    degp2 = _sc_scatter(wf_tab, row, col, ew, zeros_nh)
    xwd2, dis2 = _tc(_gcnprep_body,
                     (jax.ShapeDtypeStruct((N, H), jnp.float32),
                      jax.ShapeDtypeStruct((N, 1), jnp.float32)))(
        xw2, ones_col, jnp.eye(H, dtype=jnp.float32), degp2, wfac)
    s2 = _sc_scatter(xwd2, row, col, ew, zeros_nh)

    h2, sc2 = _tc(_hsc_body,
                  (jax.ShapeDtypeStruct((N, H), jnp.float32),
                   jax.ShapeDtypeStruct((N, 1), jnp.float32)))(
        s2, xwd2, dis2, wfac, b2[None, :], p2[:, None])
    logits, act = _tc(_pool2_body,
                      (jax.ShapeDtypeStruct((1, H), jnp.float32),
                       jax.ShapeDtypeStruct((1, H), jnp.float32)))(
        h2, sc2, wfac, gW, gb, out1,
        pW1, pb1[None, :], pW2, pb2[None, :])
    return (logits, act)


# speedup vs baseline: 11.8500x; 1.0793x over previous
import functools
import numpy as np
import jax
import jax.numpy as jnp
from jax import lax
from jax.experimental import pallas as pl
from jax.experimental.pallas import tpu as pltpu
from jax.experimental.pallas import tpu_sc as plsc

N = 10000
E = 320000
H = 128
K1 = 5000
K2 = 2500
NPAD = 10240
ROWS = NPAD // 128
NEG_INF = np.float32(-np.inf)
I32_MIN = np.int32(-(2 ** 31))

NC, NS, LANES = 2, 16, 16
NW = NC * NS
EPW = E // NW
ECH = 128
NFULL = EPW // ECH
ETAIL = EPW - NFULL * ECH
NODES_PW = NPAD // NW
NCHUNK = 8
IDXC = NCHUNK * 16
L_TOK = 16


def _mesh():
    return plsc.VectorSubcoreMesh(core_axis_name="c", subcore_axis_name="s",
                                  num_cores=NC, num_subcores=NS)


def _sc_embed_body(emb_hbm, xflat_hbm, h0_hbm, idx_v, rows_v, hbuf_v, sem):
    wid = lax.axis_index("s") * NC + lax.axis_index("c")
    nbase = wid * NODES_PW

    def chunk(ci, carry):
        node0 = nbase + ci * NCHUNK
        pltpu.sync_copy(xflat_hbm.at[pl.ds(node0 * L_TOK, IDXC)], idx_v)
        pltpu.async_copy(emb_hbm.at[idx_v], rows_v, sem).wait()

        def node(n, carry2):
            for j in range(H // 16):
                acc = rows_v[n * L_TOK, pl.ds(j * 16, 16)]
                for l in range(1, L_TOK):
                    acc = acc + rows_v[n * L_TOK + l, pl.ds(j * 16, 16)]
                hbuf_v[n, pl.ds(j * 16, 16)] = acc
            return carry2

        lax.fori_loop(0, NCHUNK, node, 0)
        pltpu.sync_copy(hbuf_v, h0_hbm.at[pl.ds(node0, NCHUNK)])
        return carry

    lax.fori_loop(0, NPAD // NW // NCHUNK, chunk, 0)


def _sc_embed(embz, xflat):
    return pl.kernel(
        _sc_embed_body,
        out_type=jax.ShapeDtypeStruct((NPAD, H), jnp.float32),
        mesh=_mesh(),
        scratch_types=[
            pltpu.VMEM((IDXC,), jnp.int32),
            pltpu.VMEM((IDXC, H), jnp.float32),
            pltpu.VMEM((NCHUNK, H), jnp.float32),
            pltpu.SemaphoreType.DMA,
        ],
    )(embz, xflat)


def _lane_splat(v16, lane):
    dnums = lax.GatherDimensionNumbers(offset_dims=(),
                                       collapsed_slice_dims=(0,),
                                       start_index_map=(0,))
    idx = jnp.full((16, 1), lane, jnp.int32)
    return lax.gather(v16, idx, dnums, slice_sizes=(1,),
                      mode=lax.GatherScatterMode.PROMISE_IN_BOUNDS)


def _sc_deg1_body(col_hbm, ew_hbm, zeros_hbm, out_hbm,
                  cidx_v, ew_v, rows_v,
                  cidx_t, ew_t, rows_t,
                  acc_sh, sem):
    cid = lax.axis_index("c")
    sid = lax.axis_index("s")
    wid = sid * NC + cid
    ebase = wid * EPW
    nps = NPAD // NS
    pltpu.sync_copy(zeros_hbm.at[pl.ds(0, ECH)], rows_v)
    pltpu.sync_copy(zeros_hbm.at[pl.ds(0, ETAIL)], rows_t)
    pltpu.sync_copy(zeros_hbm.at[pl.ds(sid * nps, nps)],
                    acc_sh.at[pl.ds(sid * nps, nps)])
    plsc.subcore_barrier()

    def process(size, cidx, ewv, rows, e0):
        pltpu.sync_copy(col_hbm.at[pl.ds(e0, size)], cidx)
        pltpu.sync_copy(ew_hbm.at[pl.ds(e0, size)], ewv)

        def fill(g, c):
            e16 = ewv[pl.ds(g * 16, 16)]
            for l in range(16):
                w16 = _lane_splat(e16, l)
                e = g * 16 + l
                for j in range(H // 16):
                    rows[e, pl.ds(j * 16, 16)] = (
                        rows[e, pl.ds(j * 16, 16)] * jnp.float32(0.0) + w16)
            return c

        lax.fori_loop(0, size // 16, fill, 0)
        pltpu.sync_copy(rows, acc_sh.at[cidx], add=True)

    def chunk(i, c):
        process(ECH, cidx_v, ew_v, rows_v, ebase + i * ECH)
        return c

    lax.fori_loop(0, NFULL, chunk, 0)
    process(ETAIL, cidx_t, ew_t, rows_t, ebase + NFULL * ECH)
    plsc.subcore_barrier()
    pltpu.sync_copy(acc_sh.at[pl.ds(sid * nps, nps)],
                    out_hbm.at[cid, pl.ds(sid * nps, nps)])


def _sc_deg1(col, ew, zeros_nh):
    return pl.kernel(
        _sc_deg1_body,
        out_type=jax.ShapeDtypeStruct((NC, NPAD, H), jnp.float32),
        mesh=_mesh(),
        scratch_types=[
            pltpu.VMEM((ECH,), jnp.int32),
            pltpu.VMEM((ECH,), jnp.float32),
            pltpu.VMEM((ECH, H), jnp.float32),
            pltpu.VMEM((ETAIL,), jnp.int32),
            pltpu.VMEM((ETAIL,), jnp.float32),
            pltpu.VMEM((ETAIL, H), jnp.float32),
            pltpu.VMEM_SHARED((NPAD, H), jnp.float32),
            pltpu.SemaphoreType.DMA,
        ],
    )(col, ew, zeros_nh)


def _sc_scatter_body(xwd_hbm, row_hbm, col_hbm, ww_hbm, zeros_hbm, out_hbm,
                     ridx_v, cidx_v, ww_v, rows_v,
                     ridx_t, cidx_t, ww_t, rows_t,
                     acc_sh, sem):
    cid = lax.axis_index("c")
    sid = lax.axis_index("s")
    wid = sid * NC + cid
    ebase = wid * EPW
    nps = NPAD // NS
    pltpu.sync_copy(zeros_hbm.at[pl.ds(sid * nps, nps)],
                    acc_sh.at[pl.ds(sid * nps, nps)])
    plsc.subcore_barrier()

    def process(size, ridx, cidx, wwv, rows, e0):
        pltpu.sync_copy(row_hbm.at[pl.ds(e0, size)], ridx)
        pltpu.sync_copy(col_hbm.at[pl.ds(e0, size)], cidx)
        pltpu.sync_copy(ww_hbm.at[pl.ds(e0, size)], wwv)
        pltpu.async_copy(xwd_hbm.at[ridx], rows, sem).wait()

        def scale(g, c):
            e16 = wwv[pl.ds(g * 16, 16)]
            for l in range(16):
                w16 = _lane_splat(e16, l)
                e = g * 16 + l
                for j in range(H // 16):
                    rows[e, pl.ds(j * 16, 16)] = (
                        rows[e, pl.ds(j * 16, 16)] * w16)
            return c

        lax.fori_loop(0, size // 16, scale, 0)
        pltpu.sync_copy(rows, acc_sh.at[cidx], add=True)

    def chunk(i, c):
        process(ECH, ridx_v, cidx_v, ww_v, rows_v, ebase + i * ECH)
        return c

    lax.fori_loop(0, NFULL, chunk, 0)
    process(ETAIL, ridx_t, cidx_t, ww_t, rows_t, ebase + NFULL * ECH)
    plsc.subcore_barrier()
    pltpu.sync_copy(acc_sh.at[pl.ds(sid * nps, nps)],
                    out_hbm.at[cid, pl.ds(sid * nps, nps)])


def _sc_scatter(xwd, row, col, ww, zeros_nh):
    return pl.kernel(
        _sc_scatter_body,
        out_type=jax.ShapeDtypeStruct((NC, NPAD, H), jnp.float32),
        mesh=_mesh(),
        scratch_types=[
            pltpu.VMEM((ECH,), jnp.int32),
            pltpu.VMEM((ECH,), jnp.int32),
            pltpu.VMEM((ECH,), jnp.float32),
            pltpu.VMEM((ECH, H), jnp.float32),
            pltpu.VMEM((ETAIL,), jnp.int32),
            pltpu.VMEM((ETAIL,), jnp.int32),
            pltpu.VMEM((ETAIL,), jnp.float32),
            pltpu.VMEM((ETAIL, H), jnp.float32),
            pltpu.VMEM_SHARED((NPAD, H), jnp.float32),
            pltpu.SemaphoreType.DMA,
        ],
    )(xwd, row, col, ww, zeros_nh)


def _tokey(f):
    b = jax.lax.bitcast_convert_type(f, jnp.int32)
    return jnp.where(b >= 0, b, (~b) ^ I32_MIN)


def _fromkey(k):
    b = jnp.where(k >= 0, k, (~k) ^ I32_MIN)
    return jax.lax.bitcast_convert_type(b, jnp.float32)


def _threshold_mask(sc, valid, k):
    scp = jnp.concatenate([jnp.where(valid, sc, NEG_INF),
                           jnp.full((NPAD - N,), NEG_INF, jnp.float32)])
    vp = jnp.concatenate([valid.astype(jnp.float32),
                          jnp.zeros((NPAD - N,), jnp.float32)]) > 0
    keys = _tokey(scp)
    kmin = jnp.min(jnp.where(vp, keys, jnp.int32(2 ** 31 - 1)))
    kmax = jnp.max(jnp.where(vp, keys, I32_MIN))

    def body(_, lohi):
        lo, hi = lohi
        mid = lo + (hi - lo + 1) // 2
        t = _fromkey(mid)
        cnt = jnp.sum(jnp.where(vp & (scp >= t), 1, 0))
        pred = cnt >= k
        return (jnp.where(pred, mid, lo), jnp.where(pred, hi, mid - 1))

    lo, _ = jax.lax.fori_loop(0, 32, body, (kmin, kmax))
    t = _fromkey(lo)
    gt = vp & (scp > t)
    c = jnp.sum(jnp.where(gt, 1, 0))
    eq = vp & (scp == t)
    eqf = eq.astype(jnp.float32).reshape(ROWS, 128)
    ii = jax.lax.broadcasted_iota(jnp.int32, (128, 128), 0)
    jj = jax.lax.broadcasted_iota(jnp.int32, (128, 128), 1)
    U = (ii <= jj).astype(jnp.float32)
    bi = jax.lax.broadcasted_iota(jnp.int32, (ROWS, ROWS), 0)
    bj = jax.lax.broadcasted_iota(jnp.int32, (ROWS, ROWS), 1)
    Ls = (bj < bi).astype(jnp.float32)
    within = jnp.dot(eqf, U, preferred_element_type=jnp.float32)
    rowsum = jnp.sum(eqf, axis=1, keepdims=True)
    prefix = jnp.dot(Ls, rowsum, preferred_element_type=jnp.float32)
    rank = (within + prefix).reshape(NPAD)
    need = (k - c).astype(jnp.float32)
    mask = gt | (eq & (rank <= need))
    return mask[:N]


def _attpool(xsel, mask, gW, gb):
    gate = jnp.dot(xsel, gW, preferred_element_type=jnp.float32)[:, 0] + gb[0]
    gate = jnp.where(mask, gate, NEG_INF)
    mx = jnp.max(gate)
    a = jnp.where(mask, jnp.exp(gate - mx), 0.0)
    a = a / jnp.sum(a)
    return jnp.dot(a[None, :], xsel, preferred_element_type=jnp.float32)


def _ew_body(attr_ref, x_ref, out_ref, inv_ref):
    a = attr_ref[...]
    ii = jax.lax.broadcasted_iota(jnp.int32, (128, 32), 0)
    jj = jax.lax.broadcasted_iota(jnp.int32, (128, 32), 1)
    Q = jnp.where((ii // 4) == jj, jnp.float32(0.25), jnp.float32(0.0))
    out_ref[...] = jnp.dot(a, Q, preferred_element_type=jnp.float32)
    cnt = jnp.sum((x_ref[...] != 0).astype(jnp.float32), axis=1,
                  keepdims=True)
    inv_ref[...] = 1.0 / jnp.maximum(cnt, 1.0)


def _gcnprep_body(h_ref, inv_ref, W_ref, degp_ref, wf_ref, xwd_ref, dis_ref):
    d3 = degp_ref[...]
    wf = wf_ref[...]
    d = wf * (d3[0, :N, 0:1] + d3[1, :N, 0:1]) + wf
    dis = jnp.where(d > 0, jax.lax.rsqrt(jnp.where(d > 0, d, 1.0)), 0.0)
    xw = jnp.dot(h_ref[...] * inv_ref[...], W_ref[...],
                 preferred_element_type=jnp.float32)
    xwd_ref[...] = xw * dis * wf
    dis_ref[...] = dis


def _hsc_body(s_ref, xwd_ref, dis_ref, wf_ref, b_ref, p_ref, h_ref, sc_ref):
    s = s_ref[0, :N, :] + s_ref[1, :N, :]
    h = jax.nn.relu(dis_ref[...] * (wf_ref[...] * s + xwd_ref[...])
                    + b_ref[...])
    p = p_ref[...]
    pn = p * jax.lax.rsqrt(jnp.sum(p * p))
    h_ref[...] = h
    sc_ref[...] = jnp.dot(h, pn, preferred_element_type=jnp.float32)


def _pool1_body(h_ref, sc_ref, gW_ref, gb_ref, W2_ref,
                xw2_ref, wfac_ref, out1_ref):
    h = h_ref[...]
    sc = sc_ref[...][:, 0]
    mask = _threshold_mask(sc, jnp.ones((N,), jnp.bool_), K1)
    wf = mask.astype(jnp.float32)[:, None]
    xsel = h * jnp.tanh(sc)[:, None] * wf
    out1_ref[...] = _attpool(xsel, mask, gW_ref[...], gb_ref[...])
    xw2_ref[...] = jnp.dot(xsel, W2_ref[...], preferred_element_type=jnp.float32)
    wfac_ref[...] = wf


def _pool2_body(h_ref, sc_ref, wfac_ref, gW_ref, gb_ref, out1_ref,
                pW1_ref, pb1_ref, pW2_ref, pb2_ref,
                logits_ref, act_ref):
    h2 = h_ref[...]
    sc2 = sc_ref[...][:, 0]
    valid = wfac_ref[...][:, 0] > 0
    mask2 = _threshold_mask(sc2, valid, K2)
    xsel2 = h2 * jnp.tanh(sc2)[:, None] * mask2.astype(jnp.float32)[:, None]
    out2 = _attpool(xsel2, mask2, gW_ref[...], gb_ref[...])
    act = out1_ref[...] + out2
    z1 = jax.nn.relu(jnp.dot(act, pW1_ref[...],
                             preferred_element_type=jnp.float32) + pb1_ref[...])
    z = jnp.dot(z1, pW2_ref[...], preferred_element_type=jnp.float32) + pb2_ref[...]
    logits_ref[...] = z * jax.lax.rsqrt(jnp.sum(z * z, axis=-1, keepdims=True))
    act_ref[...] = act


def _tc(body, out_shapes):
    return pl.pallas_call(body, out_shape=out_shapes)


def kernel(x, edge_index, edge_attr, batch, emb, W1, b1, W2, b2, p1, p2,
           gW, gb, pW1, pb1, pW2, pb2):
    row = edge_index[0]
    col = edge_index[1]

    attr2d = edge_attr.reshape(E // 32, 128)
    xpad = jnp.pad(x, ((0, NPAD - N), (0, 0)))
    ew2d, inv = _tc(_ew_body,
                    (jax.ShapeDtypeStruct((E // 32, 32), jnp.float32),
                     jax.ShapeDtypeStruct((NPAD, 1), jnp.float32)))(
        attr2d, xpad)
    ew = ew2d.reshape(E)

    embz = emb.at[0].set(0.0)
    xflat = xpad.reshape(NPAD * L_TOK)
    h0 = _sc_embed(embz, xflat)[:N]

    zeros_nh = jnp.zeros((NPAD, H), jnp.float32)
    degp1 = _sc_deg1(col, ew, zeros_nh)
    ones_col = jnp.ones((N, 1), jnp.float32)
    xwd1, dis1 = _tc(_gcnprep_body,
                     (jax.ShapeDtypeStruct((N, H), jnp.float32),
                      jax.ShapeDtypeStruct((N, 1), jnp.float32)))(
        h0, inv[:N], W1, degp1, ones_col)
    s1 = _sc_scatter(xwd1, row, col, ew, zeros_nh)

    h1, sc1 = _tc(_hsc_body,
                  (jax.ShapeDtypeStruct((N, H), jnp.float32),
                   jax.ShapeDtypeStruct((N, 1), jnp.float32)))(
        s1, xwd1, dis1, ones_col, b1[None, :], p1[:, None])
    xw2, wfac, out1 = _tc(_pool1_body,
                          (jax.ShapeDtypeStruct((N, H), jnp.float32),
                           jax.ShapeDtypeStruct((N, 1), jnp.float32),
                           jax.ShapeDtypeStruct((1, H), jnp.float32)))(
        h1, sc1, gW, gb, W2)

    wf_tab = jnp.tile(wfac, (1, H))
    degp2 = _sc_scatter(wf_tab, row, col, ew, zeros_nh)
    xwd2, dis2 = _tc(_gcnprep_body,
                     (jax.ShapeDtypeStruct((N, H), jnp.float32),
                      jax.ShapeDtypeStruct((N, 1), jnp.float32)))(
        xw2, ones_col, jnp.eye(H, dtype=jnp.float32), degp2, wfac)
    s2 = _sc_scatter(xwd2, row, col, ew, zeros_nh)

    h2, sc2 = _tc(_hsc_body,
                  (jax.ShapeDtypeStruct((N, H), jnp.float32),
                   jax.ShapeDtypeStruct((N, 1), jnp.float32)))(
        s2, xwd2, dis2, wfac, b2[None, :], p2[:, None])
    logits, act = _tc(_pool2_body,
                      (jax.ShapeDtypeStruct((1, H), jnp.float32),
                       jax.ShapeDtypeStruct((1, H), jnp.float32)))(
        h2, sc2, wfac, gW, gb, out1,
        pW1, pb1[None, :], pW2, pb2[None, :])
    return (logits, act)


# speedup vs baseline: 14.1606x; 1.1950x over previous
import functools
import numpy as np
import jax
import jax.numpy as jnp
from jax import lax
from jax.experimental import pallas as pl
from jax.experimental.pallas import tpu as pltpu
from jax.experimental.pallas import tpu_sc as plsc

N = 10000
E = 320000
H = 128
K1 = 5000
K2 = 2500
NPAD = 10240
ROWS = NPAD // 128
NEG_INF = np.float32(-np.inf)
I32_MIN = np.int32(-(2 ** 31))

NC, NS, LANES = 2, 16, 16
NW = NC * NS
EPW = E // NW
ECH = 128
NFULL = EPW // ECH
ETAIL = EPW - NFULL * ECH
NODES_PW = NPAD // NW
NCHUNK = 8
IDXC = NCHUNK * 16
L_TOK = 16


def _mesh():
    return plsc.VectorSubcoreMesh(core_axis_name="c", subcore_axis_name="s",
                                  num_cores=NC, num_subcores=NS)


def _sc_embed_body(emb_hbm, xflat_hbm, h0_hbm, idx_v, rows_v, hbuf_v, sem):
    wid = lax.axis_index("s") * NC + lax.axis_index("c")
    nbase = wid * NODES_PW

    def chunk(ci, carry):
        node0 = nbase + ci * NCHUNK
        pltpu.sync_copy(xflat_hbm.at[pl.ds(node0 * L_TOK, IDXC)], idx_v)
        pltpu.async_copy(emb_hbm.at[idx_v], rows_v, sem).wait()

        def node(n, carry2):
            for j in range(H // 16):
                acc = rows_v[n * L_TOK, pl.ds(j * 16, 16)]
                for l in range(1, L_TOK):
                    acc = acc + rows_v[n * L_TOK + l, pl.ds(j * 16, 16)]
                hbuf_v[n, pl.ds(j * 16, 16)] = acc
            return carry2

        lax.fori_loop(0, NCHUNK, node, 0)
        pltpu.sync_copy(hbuf_v, h0_hbm.at[pl.ds(node0, NCHUNK)])
        return carry

    lax.fori_loop(0, NPAD // NW // NCHUNK, chunk, 0)


def _sc_embed(embz, xflat):
    return pl.kernel(
        _sc_embed_body,
        out_type=jax.ShapeDtypeStruct((NPAD, H), jnp.float32),
        mesh=_mesh(),
        scratch_types=[
            pltpu.VMEM((IDXC,), jnp.int32),
            pltpu.VMEM((IDXC, H), jnp.float32),
            pltpu.VMEM((NCHUNK, H), jnp.float32),
            pltpu.SemaphoreType.DMA,
        ],
    )(embz, xflat)


def _lane_splat(v16, lane):
    dnums = lax.GatherDimensionNumbers(offset_dims=(),
                                       collapsed_slice_dims=(0,),
                                       start_index_map=(0,))
    idx = jnp.full((16, 1), lane, jnp.int32)
    return lax.gather(v16, idx, dnums, slice_sizes=(1,),
                      mode=lax.GatherScatterMode.PROMISE_IN_BOUNDS)


def _sc_deg1_body(col_hbm, ew_hbm, zeros_hbm, out_hbm,
                  cidx_v, ew_v, rows_v,
                  cidx_t, ew_t, rows_t,
                  acc_sh, sem):
    cid = lax.axis_index("c")
    sid = lax.axis_index("s")
    wid = sid * NC + cid
    ebase = wid * EPW
    nps = NPAD // NS
    pltpu.sync_copy(zeros_hbm.at[pl.ds(0, ECH)], rows_v)
    pltpu.sync_copy(zeros_hbm.at[pl.ds(0, ETAIL)], rows_t)
    pltpu.sync_copy(zeros_hbm.at[pl.ds(sid * nps, nps)],
                    acc_sh.at[pl.ds(sid * nps, nps)])
    plsc.subcore_barrier()

    def process(size, cidx, ewv, rows, e0):
        pltpu.sync_copy(col_hbm.at[pl.ds(e0, size)], cidx)
        pltpu.sync_copy(ew_hbm.at[pl.ds(e0, size)], ewv)

        def fill(g, c):
            e16 = ewv[pl.ds(g * 16, 16)]
            for l in range(16):
                w16 = _lane_splat(e16, l)
                e = g * 16 + l
                for j in range(H // 16):
                    rows[e, pl.ds(j * 16, 16)] = (
                        rows[e, pl.ds(j * 16, 16)] * jnp.float32(0.0) + w16)
            return c

        lax.fori_loop(0, size // 16, fill, 0)
        pltpu.sync_copy(rows, acc_sh.at[cidx], add=True)

    def chunk(i, c):
        process(ECH, cidx_v, ew_v, rows_v, ebase + i * ECH)
        return c

    lax.fori_loop(0, NFULL, chunk, 0)
    process(ETAIL, cidx_t, ew_t, rows_t, ebase + NFULL * ECH)
    plsc.subcore_barrier()
    pltpu.sync_copy(acc_sh.at[pl.ds(sid * nps, nps)],
                    out_hbm.at[cid, pl.ds(sid * nps, nps)])


def _sc_deg1(col, ew, zeros_nh):
    return pl.kernel(
        _sc_deg1_body,
        out_type=jax.ShapeDtypeStruct((NC, NPAD, H), jnp.float32),
        mesh=_mesh(),
        scratch_types=[
            pltpu.VMEM((ECH,), jnp.int32),
            pltpu.VMEM((ECH,), jnp.float32),
            pltpu.VMEM((ECH, H), jnp.float32),
            pltpu.VMEM((ETAIL,), jnp.int32),
            pltpu.VMEM((ETAIL,), jnp.float32),
            pltpu.VMEM((ETAIL, H), jnp.float32),
            pltpu.VMEM_SHARED((NPAD, H), jnp.float32),
            pltpu.SemaphoreType.DMA,
        ],
    )(col, ew, zeros_nh)


def _sc_scatter_body(xwd_hbm, row_hbm, col_hbm, ww_hbm, zeros_hbm, out_hbm,
                     ridx_a, cidx_a, ww_a, rows_a,
                     ridx_b, cidx_b, ww_b, rows_b,
                     ridx_t, cidx_t, ww_t, rows_t,
                     acc_sh, sem_a, sem_b):
    cid = lax.axis_index("c")
    sid = lax.axis_index("s")
    wid = sid * NC + cid
    ebase = wid * EPW
    nps = NPAD // NS
    pltpu.sync_copy(zeros_hbm.at[pl.ds(sid * nps, nps)],
                    acc_sh.at[pl.ds(sid * nps, nps)])
    plsc.subcore_barrier()

    def fetch(size, ridx, cidx, wwv, rows, e0, sem):
        pltpu.sync_copy(row_hbm.at[pl.ds(e0, size)], ridx)
        pltpu.sync_copy(col_hbm.at[pl.ds(e0, size)], cidx)
        pltpu.sync_copy(ww_hbm.at[pl.ds(e0, size)], wwv)
        return pltpu.async_copy(xwd_hbm.at[ridx], rows, sem)

    def drain(size, cidx, wwv, rows, desc):
        desc.wait()

        def scale(g, c):
            e16 = wwv[pl.ds(g * 16, 16)]
            for l in range(16):
                w16 = _lane_splat(e16, l)
                e = g * 16 + l
                for j in range(H // 16):
                    rows[e, pl.ds(j * 16, 16)] = (
                        rows[e, pl.ds(j * 16, 16)] * w16)
            return c

        lax.fori_loop(0, size // 16, scale, 0)
        pltpu.sync_copy(rows, acc_sh.at[cidx], add=True)

    def pair(i, c):
        e0a = ebase + (2 * i) * ECH
        da = fetch(ECH, ridx_a, cidx_a, ww_a, rows_a, e0a, sem_a)
        db = fetch(ECH, ridx_b, cidx_b, ww_b, rows_b, e0a + ECH, sem_b)
        drain(ECH, cidx_a, ww_a, rows_a, da)
        drain(ECH, cidx_b, ww_b, rows_b, db)
        return c

    lax.fori_loop(0, NFULL // 2, pair, 0)
    dt = fetch(ETAIL, ridx_t, cidx_t, ww_t, rows_t, ebase + NFULL * ECH,
               sem_a)
    drain(ETAIL, cidx_t, ww_t, rows_t, dt)
    plsc.subcore_barrier()
    pltpu.sync_copy(acc_sh.at[pl.ds(sid * nps, nps)],
                    out_hbm.at[cid, pl.ds(sid * nps, nps)])


def _sc_scatter(xwd, row, col, ww, zeros_nh):
    return pl.kernel(
        _sc_scatter_body,
        out_type=jax.ShapeDtypeStruct((NC, NPAD, H), jnp.float32),
        mesh=_mesh(),
        scratch_types=[
            pltpu.VMEM((ECH,), jnp.int32),
            pltpu.VMEM((ECH,), jnp.int32),
            pltpu.VMEM((ECH,), jnp.float32),
            pltpu.VMEM((ECH, H), jnp.float32),
            pltpu.VMEM((ECH,), jnp.int32),
            pltpu.VMEM((ECH,), jnp.int32),
            pltpu.VMEM((ECH,), jnp.float32),
            pltpu.VMEM((ECH, H), jnp.float32),
            pltpu.VMEM((ETAIL,), jnp.int32),
            pltpu.VMEM((ETAIL,), jnp.int32),
            pltpu.VMEM((ETAIL,), jnp.float32),
            pltpu.VMEM((ETAIL, H), jnp.float32),
            pltpu.VMEM_SHARED((NPAD, H), jnp.float32),
            pltpu.SemaphoreType.DMA,
            pltpu.SemaphoreType.DMA,
        ],
    )(xwd, row, col, ww, zeros_nh)


def _tokey(f):
    b = jax.lax.bitcast_convert_type(f, jnp.int32)
    return jnp.where(b >= 0, b, (~b) ^ I32_MIN)


def _fromkey(k):
    b = jnp.where(k >= 0, k, (~k) ^ I32_MIN)
    return jax.lax.bitcast_convert_type(b, jnp.float32)


def _threshold_mask(sc, valid, k):
    scp = jnp.concatenate([jnp.where(valid, sc, NEG_INF),
                           jnp.full((NPAD - N,), NEG_INF, jnp.float32)])
    vp = jnp.concatenate([valid.astype(jnp.float32),
                          jnp.zeros((NPAD - N,), jnp.float32)]) > 0
    keys = _tokey(scp)
    kmin = jnp.min(jnp.where(vp, keys, jnp.int32(2 ** 31 - 1)))
    kmax = jnp.max(jnp.where(vp, keys, I32_MIN))

    def body(_, lohi):
        lo, hi = lohi
        mid = lo + (hi - lo + 1) // 2
        t = _fromkey(mid)
        cnt = jnp.sum(jnp.where(vp & (scp >= t), 1, 0))
        pred = cnt >= k
        return (jnp.where(pred, mid, lo), jnp.where(pred, hi, mid - 1))

    lo, _ = jax.lax.fori_loop(0, 32, body, (kmin, kmax))
    t = _fromkey(lo)
    gt = vp & (scp > t)
    c = jnp.sum(jnp.where(gt, 1, 0))
    eq = vp & (scp == t)
    eqf = eq.astype(jnp.float32).reshape(ROWS, 128)
    ii = jax.lax.broadcasted_iota(jnp.int32, (128, 128), 0)
    jj = jax.lax.broadcasted_iota(jnp.int32, (128, 128), 1)
    U = (ii <= jj).astype(jnp.float32)
    bi = jax.lax.broadcasted_iota(jnp.int32, (ROWS, ROWS), 0)
    bj = jax.lax.broadcasted_iota(jnp.int32, (ROWS, ROWS), 1)
    Ls = (bj < bi).astype(jnp.float32)
    within = jnp.dot(eqf, U, preferred_element_type=jnp.float32)
    rowsum = jnp.sum(eqf, axis=1, keepdims=True)
    prefix = jnp.dot(Ls, rowsum, preferred_element_type=jnp.float32)
    rank = (within + prefix).reshape(NPAD)
    need = (k - c).astype(jnp.float32)
    mask = gt | (eq & (rank <= need))
    return mask[:N]


def _attpool(xsel, mask, gW, gb):
    gate = jnp.dot(xsel, gW, preferred_element_type=jnp.float32)[:, 0] + gb[0]
    gate = jnp.where(mask, gate, NEG_INF)
    mx = jnp.max(gate)
    a = jnp.where(mask, jnp.exp(gate - mx), 0.0)
    a = a / jnp.sum(a)
    return jnp.dot(a[None, :], xsel, preferred_element_type=jnp.float32)


def _ew_body(attr_ref, x_ref, out_ref, inv_ref):
    a = attr_ref[...]
    ii = jax.lax.broadcasted_iota(jnp.int32, (128, 32), 0)
    jj = jax.lax.broadcasted_iota(jnp.int32, (128, 32), 1)
    Q = jnp.where((ii // 4) == jj, jnp.float32(0.25), jnp.float32(0.0))
    out_ref[...] = jnp.dot(a, Q, preferred_element_type=jnp.float32)
    cnt = jnp.sum((x_ref[...] != 0).astype(jnp.float32), axis=1,
                  keepdims=True)
    inv_ref[...] = 1.0 / jnp.maximum(cnt, 1.0)


def _gcnprep_body(h_ref, inv_ref, W_ref, degp_ref, wf_ref, xwd_ref, dis_ref):
    d3 = degp_ref[...]
    wf = wf_ref[...]
    d = wf * (d3[0, :N, 0:1] + d3[1, :N, 0:1]) + wf
    dis = jnp.where(d > 0, jax.lax.rsqrt(jnp.where(d > 0, d, 1.0)), 0.0)
    xw = jnp.dot(h_ref[...] * inv_ref[...], W_ref[...],
                 preferred_element_type=jnp.float32)
    xwd_ref[...] = xw * dis * wf
    dis_ref[...] = dis


def _hsc_body(s_ref, xwd_ref, dis_ref, wf_ref, b_ref, p_ref, h_ref, sc_ref):
    s = s_ref[0, :N, :] + s_ref[1, :N, :]
    h = jax.nn.relu(dis_ref[...] * (wf_ref[...] * s + xwd_ref[...])
                    + b_ref[...])
    p = p_ref[...]
    pn = p * jax.lax.rsqrt(jnp.sum(p * p))
    h_ref[...] = h
    sc_ref[...] = jnp.dot(h, pn, preferred_element_type=jnp.float32)


def _pool1_body(h_ref, sc_ref, gW_ref, gb_ref, W2_ref,
                xw2_ref, wfac_ref, out1_ref):
    h = h_ref[...]
    sc = sc_ref[...][:, 0]
    mask = _threshold_mask(sc, jnp.ones((N,), jnp.bool_), K1)
    wf = mask.astype(jnp.float32)[:, None]
    xsel = h * jnp.tanh(sc)[:, None] * wf
    out1_ref[...] = _attpool(xsel, mask, gW_ref[...], gb_ref[...])
    xw2_ref[...] = jnp.dot(xsel, W2_ref[...], preferred_element_type=jnp.float32)
    wfac_ref[...] = wf


def _pool2_body(h_ref, sc_ref, wfac_ref, gW_ref, gb_ref, out1_ref,
                pW1_ref, pb1_ref, pW2_ref, pb2_ref,
                logits_ref, act_ref):
    h2 = h_ref[...]
    sc2 = sc_ref[...][:, 0]
    valid = wfac_ref[...][:, 0] > 0
    mask2 = _threshold_mask(sc2, valid, K2)
    xsel2 = h2 * jnp.tanh(sc2)[:, None] * mask2.astype(jnp.float32)[:, None]
    out2 = _attpool(xsel2, mask2, gW_ref[...], gb_ref[...])
    act = out1_ref[...] + out2
    z1 = jax.nn.relu(jnp.dot(act, pW1_ref[...],
                             preferred_element_type=jnp.float32) + pb1_ref[...])
    z = jnp.dot(z1, pW2_ref[...], preferred_element_type=jnp.float32) + pb2_ref[...]
    logits_ref[...] = z * jax.lax.rsqrt(jnp.sum(z * z, axis=-1, keepdims=True))
    act_ref[...] = act


def _tc(body, out_shapes):
    return pl.pallas_call(body, out_shape=out_shapes)


def kernel(x, edge_index, edge_attr, batch, emb, W1, b1, W2, b2, p1, p2,
           gW, gb, pW1, pb1, pW2, pb2):
    row = edge_index[0]
    col = edge_index[1]

    attr2d = edge_attr.reshape(E // 32, 128)
    xpad = jnp.pad(x, ((0, NPAD - N), (0, 0)))
    ew2d, inv = _tc(_ew_body,
                    (jax.ShapeDtypeStruct((E // 32, 32), jnp.float32),
                     jax.ShapeDtypeStruct((NPAD, 1), jnp.float32)))(
        attr2d, xpad)
    ew = ew2d.reshape(E)

    embz = emb.at[0].set(0.0)
    xflat = xpad.reshape(NPAD * L_TOK)
    h0 = _sc_embed(embz, xflat)[:N]

    zeros_nh = jnp.zeros((NPAD, H), jnp.float32)
    degp1 = _sc_deg1(col, ew, zeros_nh)
    ones_col = jnp.ones((N, 1), jnp.float32)
    xwd1, dis1 = _tc(_gcnprep_body,
                     (jax.ShapeDtypeStruct((N, H), jnp.float32),
                      jax.ShapeDtypeStruct((N, 1), jnp.float32)))(
        h0, inv[:N], W1, degp1, ones_col)
    s1 = _sc_scatter(xwd1, row, col, ew, zeros_nh)

    h1, sc1 = _tc(_hsc_body,
                  (jax.ShapeDtypeStruct((N, H), jnp.float32),
                   jax.ShapeDtypeStruct((N, 1), jnp.float32)))(
        s1, xwd1, dis1, ones_col, b1[None, :], p1[:, None])
    xw2, wfac, out1 = _tc(_pool1_body,
                          (jax.ShapeDtypeStruct((N, H), jnp.float32),
                           jax.ShapeDtypeStruct((N, 1), jnp.float32),
                           jax.ShapeDtypeStruct((1, H), jnp.float32)))(
        h1, sc1, gW, gb, W2)

    wf_tab = jnp.tile(wfac, (1, H))
    degp2 = _sc_scatter(wf_tab, row, col, ew, zeros_nh)
    xwd2, dis2 = _tc(_gcnprep_body,
                     (jax.ShapeDtypeStruct((N, H), jnp.float32),
                      jax.ShapeDtypeStruct((N, 1), jnp.float32)))(
        xw2, ones_col, jnp.eye(H, dtype=jnp.float32), degp2, wfac)
    s2 = _sc_scatter(xwd2, row, col, ew, zeros_nh)

    h2, sc2 = _tc(_hsc_body,
                  (jax.ShapeDtypeStruct((N, H), jnp.float32),
                   jax.ShapeDtypeStruct((N, 1), jnp.float32)))(
        s2, xwd2, dis2, wfac, b2[None, :], p2[:, None])
    logits, act = _tc(_pool2_body,
                      (jax.ShapeDtypeStruct((1, H), jnp.float32),
                       jax.ShapeDtypeStruct((1, H), jnp.float32)))(
        h2, sc2, wfac, gW, gb, out1,
        pW1, pb1[None, :], pW2, pb2[None, :])
    return (logits, act)


# speedup vs baseline: 14.2533x; 1.0065x over previous
import functools
import numpy as np
import jax
import jax.numpy as jnp
from jax import lax
from jax.experimental import pallas as pl
from jax.experimental.pallas import tpu as pltpu
from jax.experimental.pallas import tpu_sc as plsc

N = 10000
E = 320000
H = 128
K1 = 5000
K2 = 2500
NPAD = 10240
ROWS = NPAD // 128
NEG_INF = np.float32(-np.inf)
I32_MIN = np.int32(-(2 ** 31))

NC, NS, LANES = 2, 16, 16
NW = NC * NS
EPW = E // NW
ECH = 128
NFULL = EPW // ECH
ETAIL = EPW - NFULL * ECH
NODES_PW = NPAD // NW
NCHUNK = 8
IDXC = NCHUNK * 16
L_TOK = 16


def _mesh():
    return plsc.VectorSubcoreMesh(core_axis_name="c", subcore_axis_name="s",
                                  num_cores=NC, num_subcores=NS)


def _sc_embed_body(emb_hbm, xflat_hbm, h0_hbm,
                   idx_a, rows_a, hbuf_a, idx_b, rows_b, hbuf_b,
                   sem_a, sem_b):
    wid = lax.axis_index("s") * NC + lax.axis_index("c")
    nbase = wid * NODES_PW

    def fetch(node0, idx_v, rows_v, sem):
        pltpu.sync_copy(xflat_hbm.at[pl.ds(node0 * L_TOK, IDXC)], idx_v)
        return pltpu.async_copy(emb_hbm.at[idx_v], rows_v, sem)

    def drain(node0, rows_v, hbuf_v, desc):
        desc.wait()

        def node(n, carry2):
            for j in range(H // 16):
                acc = rows_v[n * L_TOK, pl.ds(j * 16, 16)]
                for l in range(1, L_TOK):
                    acc = acc + rows_v[n * L_TOK + l, pl.ds(j * 16, 16)]
                hbuf_v[n, pl.ds(j * 16, 16)] = acc
            return carry2

        lax.fori_loop(0, NCHUNK, node, 0)
        pltpu.sync_copy(hbuf_v, h0_hbm.at[pl.ds(node0, NCHUNK)])

    def pair(ci, carry):
        n0a = nbase + (2 * ci) * NCHUNK
        n0b = n0a + NCHUNK
        da = fetch(n0a, idx_a, rows_a, sem_a)
        db = fetch(n0b, idx_b, rows_b, sem_b)
        drain(n0a, rows_a, hbuf_a, da)
        drain(n0b, rows_b, hbuf_b, db)
        return carry

    lax.fori_loop(0, NPAD // NW // NCHUNK // 2, pair, 0)


def _sc_embed(embz, xflat):
    return pl.kernel(
        _sc_embed_body,
        out_type=jax.ShapeDtypeStruct((NPAD, H), jnp.float32),
        mesh=_mesh(),
        scratch_types=[
            pltpu.VMEM((IDXC,), jnp.int32),
            pltpu.VMEM((IDXC, H), jnp.float32),
            pltpu.VMEM((NCHUNK, H), jnp.float32),
            pltpu.VMEM((IDXC,), jnp.int32),
            pltpu.VMEM((IDXC, H), jnp.float32),
            pltpu.VMEM((NCHUNK, H), jnp.float32),
            pltpu.SemaphoreType.DMA,
            pltpu.SemaphoreType.DMA,
        ],
    )(embz, xflat)


def _lane_splat(v16, lane):
    dnums = lax.GatherDimensionNumbers(offset_dims=(),
                                       collapsed_slice_dims=(0,),
                                       start_index_map=(0,))
    idx = jnp.full((16, 1), lane, jnp.int32)
    return lax.gather(v16, idx, dnums, slice_sizes=(1,),
                      mode=lax.GatherScatterMode.PROMISE_IN_BOUNDS)


def _sc_deg1_body(col_hbm, ew_hbm, zeros_hbm, out_hbm,
                  cidx_v, ew_v, rows_v,
                  cidx_t, ew_t, rows_t,
                  acc_sh, sem):
    cid = lax.axis_index("c")
    sid = lax.axis_index("s")
    wid = sid * NC + cid
    ebase = wid * EPW
    nps = NPAD // NS
    pltpu.sync_copy(zeros_hbm.at[pl.ds(0, ECH)], rows_v)
    pltpu.sync_copy(zeros_hbm.at[pl.ds(0, ETAIL)], rows_t)
    pltpu.sync_copy(zeros_hbm.at[pl.ds(sid * nps, nps)],
                    acc_sh.at[pl.ds(sid * nps, nps)])
    plsc.subcore_barrier()

    def process(size, cidx, ewv, rows, e0):
        pltpu.sync_copy(col_hbm.at[pl.ds(e0, size)], cidx)
        pltpu.sync_copy(ew_hbm.at[pl.ds(e0, size)], ewv)

        def fill(g, c):
            e16 = ewv[pl.ds(g * 16, 16)]
            for l in range(16):
                w16 = _lane_splat(e16, l)
                e = g * 16 + l
                for j in range(H // 16):
                    rows[e, pl.ds(j * 16, 16)] = (
                        rows[e, pl.ds(j * 16, 16)] * jnp.float32(0.0) + w16)
            return c

        lax.fori_loop(0, size // 16, fill, 0)
        pltpu.sync_copy(rows, acc_sh.at[cidx], add=True)

    def chunk(i, c):
        process(ECH, cidx_v, ew_v, rows_v, ebase + i * ECH)
        return c

    lax.fori_loop(0, NFULL, chunk, 0)
    process(ETAIL, cidx_t, ew_t, rows_t, ebase + NFULL * ECH)
    plsc.subcore_barrier()
    pltpu.sync_copy(acc_sh.at[pl.ds(sid * nps, nps)],
                    out_hbm.at[cid, pl.ds(sid * nps, nps)])


def _sc_deg1(col, ew, zeros_nh):
    return pl.kernel(
        _sc_deg1_body,
        out_type=jax.ShapeDtypeStruct((NC, NPAD, H), jnp.float32),
        mesh=_mesh(),
        scratch_types=[
            pltpu.VMEM((ECH,), jnp.int32),
            pltpu.VMEM((ECH,), jnp.float32),
            pltpu.VMEM((ECH, H), jnp.float32),
            pltpu.VMEM((ETAIL,), jnp.int32),
            pltpu.VMEM((ETAIL,), jnp.float32),
            pltpu.VMEM((ETAIL, H), jnp.float32),
            pltpu.VMEM_SHARED((NPAD, H), jnp.float32),
            pltpu.SemaphoreType.DMA,
        ],
    )(col, ew, zeros_nh)


def _sc_scatter_body(xwd_hbm, row_hbm, col_hbm, ww_hbm, zeros_hbm, out_hbm,
                     ridx_a, cidx_a, ww_a, rows_a,
                     ridx_b, cidx_b, ww_b, rows_b,
                     ridx_t, cidx_t, ww_t, rows_t,
                     acc_sh, sem_a, sem_b):
    cid = lax.axis_index("c")
    sid = lax.axis_index("s")
    wid = sid * NC + cid
    ebase = wid * EPW
    nps = NPAD // NS
    pltpu.sync_copy(zeros_hbm.at[pl.ds(sid * nps, nps)],
                    acc_sh.at[pl.ds(sid * nps, nps)])
    plsc.subcore_barrier()

    def fetch(size, ridx, cidx, wwv, rows, e0, sem):
        pltpu.sync_copy(row_hbm.at[pl.ds(e0, size)], ridx)
        pltpu.sync_copy(col_hbm.at[pl.ds(e0, size)], cidx)
        pltpu.sync_copy(ww_hbm.at[pl.ds(e0, size)], wwv)
        return pltpu.async_copy(xwd_hbm.at[ridx], rows, sem)

    def drain(size, cidx, wwv, rows, desc):
        desc.wait()

        def scale(g, c):
            e16 = wwv[pl.ds(g * 16, 16)]
            for l in range(16):
                w16 = _lane_splat(e16, l)
                e = g * 16 + l
                for j in range(H // 16):
                    rows[e, pl.ds(j * 16, 16)] = (
                        rows[e, pl.ds(j * 16, 16)] * w16)
            return c

        lax.fori_loop(0, size // 16, scale, 0)
        pltpu.sync_copy(rows, acc_sh.at[cidx], add=True)

    def pair(i, c):
        e0a = ebase + (2 * i) * ECH
        da = fetch(ECH, ridx_a, cidx_a, ww_a, rows_a, e0a, sem_a)
        db = fetch(ECH, ridx_b, cidx_b, ww_b, rows_b, e0a + ECH, sem_b)
        drain(ECH, cidx_a, ww_a, rows_a, da)
        drain(ECH, cidx_b, ww_b, rows_b, db)
        return c

    lax.fori_loop(0, NFULL // 2, pair, 0)
    dt = fetch(ETAIL, ridx_t, cidx_t, ww_t, rows_t, ebase + NFULL * ECH,
               sem_a)
    drain(ETAIL, cidx_t, ww_t, rows_t, dt)
    plsc.subcore_barrier()
    pltpu.sync_copy(acc_sh.at[pl.ds(sid * nps, nps)],
                    out_hbm.at[cid, pl.ds(sid * nps, nps)])


def _sc_scatter(xwd, row, col, ww, zeros_nh):
    return pl.kernel(
        _sc_scatter_body,
        out_type=jax.ShapeDtypeStruct((NC, NPAD, H), jnp.float32),
        mesh=_mesh(),
        scratch_types=[
            pltpu.VMEM((ECH,), jnp.int32),
            pltpu.VMEM((ECH,), jnp.int32),
            pltpu.VMEM((ECH,), jnp.float32),
            pltpu.VMEM((ECH, H), jnp.float32),
            pltpu.VMEM((ECH,), jnp.int32),
            pltpu.VMEM((ECH,), jnp.int32),
            pltpu.VMEM((ECH,), jnp.float32),
            pltpu.VMEM((ECH, H), jnp.float32),
            pltpu.VMEM((ETAIL,), jnp.int32),
            pltpu.VMEM((ETAIL,), jnp.int32),
            pltpu.VMEM((ETAIL,), jnp.float32),
            pltpu.VMEM((ETAIL, H), jnp.float32),
            pltpu.VMEM_SHARED((NPAD, H), jnp.float32),
            pltpu.SemaphoreType.DMA,
            pltpu.SemaphoreType.DMA,
        ],
    )(xwd, row, col, ww, zeros_nh)


def _tokey(f):
    b = jax.lax.bitcast_convert_type(f, jnp.int32)
    return jnp.where(b >= 0, b, (~b) ^ I32_MIN)


def _fromkey(k):
    b = jnp.where(k >= 0, k, (~k) ^ I32_MIN)
    return jax.lax.bitcast_convert_type(b, jnp.float32)


def _threshold_mask(sc, valid, k):
    scp = jnp.concatenate([jnp.where(valid, sc, NEG_INF),
                           jnp.full((NPAD - N,), NEG_INF, jnp.float32)])
    vp = jnp.concatenate([valid.astype(jnp.float32),
                          jnp.zeros((NPAD - N,), jnp.float32)]) > 0
    keys = _tokey(scp)
    kmin = jnp.min(jnp.where(vp, keys, jnp.int32(2 ** 31 - 1)))
    kmax = jnp.max(jnp.where(vp, keys, I32_MIN))

    def body(_, lohi):
        lo, hi = lohi
        mid = lo + (hi - lo + 1) // 2
        t = _fromkey(mid)
        cnt = jnp.sum(jnp.where(vp & (scp >= t), 1, 0))
        pred = cnt >= k
        return (jnp.where(pred, mid, lo), jnp.where(pred, hi, mid - 1))

    lo, _ = jax.lax.fori_loop(0, 32, body, (kmin, kmax))
    t = _fromkey(lo)
    gt = vp & (scp > t)
    c = jnp.sum(jnp.where(gt, 1, 0))
    eq = vp & (scp == t)
    eqf = eq.astype(jnp.float32).reshape(ROWS, 128)
    ii = jax.lax.broadcasted_iota(jnp.int32, (128, 128), 0)
    jj = jax.lax.broadcasted_iota(jnp.int32, (128, 128), 1)
    U = (ii <= jj).astype(jnp.float32)
    bi = jax.lax.broadcasted_iota(jnp.int32, (ROWS, ROWS), 0)
    bj = jax.lax.broadcasted_iota(jnp.int32, (ROWS, ROWS), 1)
    Ls = (bj < bi).astype(jnp.float32)
    within = jnp.dot(eqf, U, preferred_element_type=jnp.float32)
    rowsum = jnp.sum(eqf, axis=1, keepdims=True)
    prefix = jnp.dot(Ls, rowsum, preferred_element_type=jnp.float32)
    rank = (within + prefix).reshape(NPAD)
    need = (k - c).astype(jnp.float32)
    mask = gt | (eq & (rank <= need))
    return mask[:N]


def _attpool(xsel, mask, gW, gb):
    gate = jnp.dot(xsel, gW, preferred_element_type=jnp.float32)[:, 0] + gb[0]
    gate = jnp.where(mask, gate, NEG_INF)
    mx = jnp.max(gate)
    a = jnp.where(mask, jnp.exp(gate - mx), 0.0)
    a = a / jnp.sum(a)
    return jnp.dot(a[None, :], xsel, preferred_element_type=jnp.float32)


def _ew_body(attr_ref, x_ref, out_ref, inv_ref):
    a = attr_ref[...]
    ii = jax.lax.broadcasted_iota(jnp.int32, (128, 32), 0)
    jj = jax.lax.broadcasted_iota(jnp.int32, (128, 32), 1)
    Q = jnp.where((ii // 4) == jj, jnp.float32(0.25), jnp.float32(0.0))
    out_ref[...] = jnp.dot(a, Q, preferred_element_type=jnp.float32)
    cnt = jnp.sum((x_ref[...] != 0).astype(jnp.float32), axis=1,
                  keepdims=True)
    inv_ref[...] = 1.0 / jnp.maximum(cnt, 1.0)


def _gcnprep_body(h_ref, inv_ref, W_ref, degp_ref, wf_ref, xwd_ref, dis_ref):
    d3 = degp_ref[...]
    wf = wf_ref[...]
    d = wf * (d3[0, :N, 0:1] + d3[1, :N, 0:1]) + wf
    dis = jnp.where(d > 0, jax.lax.rsqrt(jnp.where(d > 0, d, 1.0)), 0.0)
    xw = jnp.dot(h_ref[...] * inv_ref[...], W_ref[...],
                 preferred_element_type=jnp.float32)
    xwd_ref[...] = xw * dis * wf
    dis_ref[...] = dis


def _hsc_body(s_ref, xwd_ref, dis_ref, wf_ref, b_ref, p_ref, h_ref, sc_ref):
    s = s_ref[0, :N, :] + s_ref[1, :N, :]
    h = jax.nn.relu(dis_ref[...] * (wf_ref[...] * s + xwd_ref[...])
                    + b_ref[...])
    p = p_ref[...]
    pn = p * jax.lax.rsqrt(jnp.sum(p * p))
    h_ref[...] = h
    sc_ref[...] = jnp.dot(h, pn, preferred_element_type=jnp.float32)


def _pool1_body(h_ref, sc_ref, gW_ref, gb_ref, W2_ref,
                xw2_ref, wfac_ref, out1_ref):
    h = h_ref[...]
    sc = sc_ref[...][:, 0]
    mask = _threshold_mask(sc, jnp.ones((N,), jnp.bool_), K1)
    wf = mask.astype(jnp.float32)[:, None]
    xsel = h * jnp.tanh(sc)[:, None] * wf
    out1_ref[...] = _attpool(xsel, mask, gW_ref[...], gb_ref[...])
    xw2_ref[...] = jnp.dot(xsel, W2_ref[...], preferred_element_type=jnp.float32)
    wfac_ref[...] = wf


def _pool2_body(h_ref, sc_ref, wfac_ref, gW_ref, gb_ref, out1_ref,
                pW1_ref, pb1_ref, pW2_ref, pb2_ref,
                logits_ref, act_ref):
    h2 = h_ref[...]
    sc2 = sc_ref[...][:, 0]
    valid = wfac_ref[...][:, 0] > 0
    mask2 = _threshold_mask(sc2, valid, K2)
    xsel2 = h2 * jnp.tanh(sc2)[:, None] * mask2.astype(jnp.float32)[:, None]
    out2 = _attpool(xsel2, mask2, gW_ref[...], gb_ref[...])
    act = out1_ref[...] + out2
    z1 = jax.nn.relu(jnp.dot(act, pW1_ref[...],
                             preferred_element_type=jnp.float32) + pb1_ref[...])
    z = jnp.dot(z1, pW2_ref[...], preferred_element_type=jnp.float32) + pb2_ref[...]
    logits_ref[...] = z * jax.lax.rsqrt(jnp.sum(z * z, axis=-1, keepdims=True))
    act_ref[...] = act


def _tc(body, out_shapes):
    return pl.pallas_call(body, out_shape=out_shapes)


def kernel(x, edge_index, edge_attr, batch, emb, W1, b1, W2, b2, p1, p2,
           gW, gb, pW1, pb1, pW2, pb2):
    row = edge_index[0]
    col = edge_index[1]

    attr2d = edge_attr.reshape(E // 32, 128)
    xpad = jnp.pad(x, ((0, NPAD - N), (0, 0)))
    ew2d, inv = _tc(_ew_body,
                    (jax.ShapeDtypeStruct((E // 32, 32), jnp.float32),
                     jax.ShapeDtypeStruct((NPAD, 1), jnp.float32)))(
        attr2d, xpad)
    ew = ew2d.reshape(E)

    embz = emb.at[0].set(0.0)
    xflat = xpad.reshape(NPAD * L_TOK)
    h0 = _sc_embed(embz, xflat)[:N]

    zeros_nh = jnp.zeros((NPAD, H), jnp.float32)
    degp1 = _sc_deg1(col, ew, zeros_nh)
    ones_col = jnp.ones((N, 1), jnp.float32)
    xwd1, dis1 = _tc(_gcnprep_body,
                     (jax.ShapeDtypeStruct((N, H), jnp.float32),
                      jax.ShapeDtypeStruct((N, 1), jnp.float32)))(
        h0, inv[:N], W1, degp1, ones_col)
    s1 = _sc_scatter(xwd1, row, col, ew, zeros_nh)

    h1, sc1 = _tc(_hsc_body,
                  (jax.ShapeDtypeStruct((N, H), jnp.float32),
                   jax.ShapeDtypeStruct((N, 1), jnp.float32)))(
        s1, xwd1, dis1, ones_col, b1[None, :], p1[:, None])
    xw2, wfac, out1 = _tc(_pool1_body,
                          (jax.ShapeDtypeStruct((N, H), jnp.float32),
                           jax.ShapeDtypeStruct((N, 1), jnp.float32),
                           jax.ShapeDtypeStruct((1, H), jnp.float32)))(
        h1, sc1, gW, gb, W2)

    wf_tab = jnp.tile(wfac, (1, H))
    degp2 = _sc_scatter(wf_tab, row, col, ew, zeros_nh)
    xwd2, dis2 = _tc(_gcnprep_body,
                     (jax.ShapeDtypeStruct((N, H), jnp.float32),
                      jax.ShapeDtypeStruct((N, 1), jnp.float32)))(
        xw2, ones_col, jnp.eye(H, dtype=jnp.float32), degp2, wfac)
    s2 = _sc_scatter(xwd2, row, col, ew, zeros_nh)

    h2, sc2 = _tc(_hsc_body,
                  (jax.ShapeDtypeStruct((N, H), jnp.float32),
                   jax.ShapeDtypeStruct((N, 1), jnp.float32)))(
        s2, xwd2, dis2, wfac, b2[None, :], p2[:, None])
    logits, act = _tc(_pool2_body,
                      (jax.ShapeDtypeStruct((1, H), jnp.float32),
                       jax.ShapeDtypeStruct((1, H), jnp.float32)))(
        h2, sc2, wfac, gW, gb, out1,
        pW1, pb1[None, :], pW2, pb2[None, :])
    return (logits, act)
